# baseline (device time: 575892 ns/iter reference)
import jax
import jax.numpy as jnp
from jax import lax
from jax.experimental import pallas as pl
from jax.experimental.pallas import tpu as pltpu

N_DEV = 32


def kernel(x, router_W, route_idx, expert_W):
    T, D = x.shape
    E_LOC, _, H = expert_W.shape
    E = N_DEV * E_LOC

    def body(x_ref, rw_ref, idx_ref, ew_ref, out_ref,
             xbf_ref, w_ref, comm_ref, send_sems, recv_sems, credit_sem):
        my = lax.axis_index("i")
        left = lax.rem(my - 1 + N_DEV, N_DEV)
        right = lax.rem(my + 1, N_DEV)

        barrier_sem = pltpu.get_barrier_semaphore()
        for nbr in (left, right):
            pl.semaphore_signal(
                barrier_sem, inc=1,
                device_id=(nbr,), device_id_type=pl.DeviceIdType.MESH,
            )
        pl.semaphore_wait(barrier_sem, 2)

        scores = jnp.dot(x_ref[:, :], rw_ref[:, :],
                         preferred_element_type=jnp.float32)
        e_iota = lax.broadcasted_iota(jnp.int32, (T, E), 1)
        oh0 = (e_iota == idx_ref[:, 0:1]).astype(jnp.float32)
        oh1 = (e_iota == idx_ref[:, 1:2]).astype(jnp.float32)
        s0 = jnp.sum(scores * oh0, axis=1, keepdims=True)
        s1 = jnp.sum(scores * oh1, axis=1, keepdims=True)
        m = jnp.maximum(s0, s1)
        g0 = jnp.exp(s0 - m)
        g1 = jnp.exp(s1 - m)
        den = g0 + g1
        w_ref[:, :] = oh0 * (g0 / den) + oh1 * (g1 / den)

        xbf_ref[:, :] = x_ref[:, :].astype(jnp.bfloat16)
        comm_ref[0, :, :, :] = ew_ref[:, :, :].astype(jnp.bfloat16)
        out_ref[:, :] = jnp.zeros((T, H), jnp.float32)

        def compute(slot, h):
            o = lax.rem(my - h + N_DEV, N_DEV)
            r_iota = lax.broadcasted_iota(jnp.int32, (E, E_LOC), 0)
            c_iota = lax.broadcasted_iota(jnp.int32, (E, E_LOC), 1)
            sel = (r_iota == E_LOC * o + c_iota).astype(jnp.float32)
            wb = jnp.dot(w_ref[:, :], sel,
                         preferred_element_type=jnp.float32)
            for k in range(E_LOC):
                part = jnp.dot(xbf_ref[:, :], comm_ref[slot, k],
                               preferred_element_type=jnp.float32)
                out_ref[:, :] += wb[:, k:k + 1] * part

        for h in range(N_DEV - 1):
            cur = h % 2
            nxt = (h + 1) % 2
            if h >= 1:
                pl.semaphore_wait(credit_sem, 1)
            rdma = pltpu.make_async_remote_copy(
                src_ref=comm_ref.at[cur],
                dst_ref=comm_ref.at[nxt],
                send_sem=send_sems.at[cur],
                recv_sem=recv_sems.at[nxt],
                device_id=(right,),
                device_id_type=pl.DeviceIdType.MESH,
            )
            rdma.start()
            compute(cur, h)
            rdma.wait()
            pl.semaphore_signal(
                credit_sem, inc=1,
                device_id=(left,), device_id_type=pl.DeviceIdType.MESH,
            )
        compute((N_DEV - 1) % 2, N_DEV - 1)
        pl.semaphore_wait(credit_sem, 1)

    return pl.pallas_call(
        body,
        out_shape=jax.ShapeDtypeStruct((T, H), jnp.float32),
        in_specs=[pl.BlockSpec(memory_space=pltpu.VMEM)] * 4,
        out_specs=pl.BlockSpec(memory_space=pltpu.VMEM),
        scratch_shapes=[
            pltpu.VMEM((T, D), jnp.bfloat16),
            pltpu.VMEM((T, E), jnp.float32),
            pltpu.VMEM((2, E_LOC, D, H), jnp.bfloat16),
            pltpu.SemaphoreType.DMA((2,)),
            pltpu.SemaphoreType.DMA((2,)),
            pltpu.SemaphoreType.REGULAR,
        ],
        compiler_params=pltpu.CompilerParams(collective_id=0),
    )(x, router_W, route_idx, expert_W)


# device time: 400365 ns/iter; 1.4384x vs baseline; 1.4384x over previous
import jax
import jax.numpy as jnp
from jax import lax
from jax.experimental import pallas as pl
from jax.experimental.pallas import tpu as pltpu

N_DEV = 32
A_HOPS = N_DEV // 2
B_HOPS = N_DEV - 1 - A_HOPS


def kernel(x, router_W, route_idx, expert_W):
    T, D = x.shape
    E_LOC, _, H = expert_W.shape
    E = N_DEV * E_LOC

    def body(x_ref, rw_ref, idx_ref, ew_ref, out_ref,
             xbf_ref, w_ref, comm_a, comm_b,
             send_a, recv_a, send_b, recv_b, credit_a, credit_b):
        my = lax.axis_index("i")
        left = lax.rem(my - 1 + N_DEV, N_DEV)
        right = lax.rem(my + 1, N_DEV)

        barrier_sem = pltpu.get_barrier_semaphore()
        for nbr in (left, right):
            pl.semaphore_signal(
                barrier_sem, inc=1,
                device_id=(nbr,), device_id_type=pl.DeviceIdType.MESH,
            )
        pl.semaphore_wait(barrier_sem, 2)

        scores = jnp.dot(x_ref[:, :], rw_ref[:, :],
                         preferred_element_type=jnp.float32)
        e_iota = lax.broadcasted_iota(jnp.int32, (T, E), 1)
        oh0 = (e_iota == idx_ref[:, 0:1]).astype(jnp.float32)
        oh1 = (e_iota == idx_ref[:, 1:2]).astype(jnp.float32)
        s0 = jnp.sum(scores * oh0, axis=1, keepdims=True)
        s1 = jnp.sum(scores * oh1, axis=1, keepdims=True)
        m = jnp.maximum(s0, s1)
        g0 = jnp.exp(s0 - m)
        g1 = jnp.exp(s1 - m)
        den = g0 + g1
        w_ref[:, :] = oh0 * (g0 / den) + oh1 * (g1 / den)

        xbf_ref[:, :] = x_ref[:, :].astype(jnp.bfloat16)
        blk = ew_ref[:, :, :].astype(jnp.bfloat16)
        comm_a[0, :, :, :] = blk
        comm_b[0, :, :, :] = blk
        out_ref[:, :] = jnp.zeros((T, H), jnp.float32)

        def compute(comm_ref, slot, origin):
            r_iota = lax.broadcasted_iota(jnp.int32, (E, E_LOC), 0)
            c_iota = lax.broadcasted_iota(jnp.int32, (E, E_LOC), 1)
            sel = (r_iota == E_LOC * origin + c_iota).astype(jnp.float32)
            wb = jnp.dot(w_ref[:, :], sel,
                         preferred_element_type=jnp.float32)
            for k in range(E_LOC):
                part = jnp.dot(xbf_ref[:, :], comm_ref[slot, k],
                               preferred_element_type=jnp.float32)
                out_ref[:, :] += wb[:, k:k + 1] * part

        for h in range(A_HOPS):
            cur = h % 2
            nxt = (h + 1) % 2
            b_active = h < B_HOPS

            if h >= 1:
                pl.semaphore_wait(credit_a, 1)
            rdma_a = pltpu.make_async_remote_copy(
                src_ref=comm_a.at[cur], dst_ref=comm_a.at[nxt],
                send_sem=send_a.at[cur], recv_sem=recv_a.at[nxt],
                device_id=(right,), device_id_type=pl.DeviceIdType.MESH,
            )
            rdma_a.start()

            if b_active:
                if h >= 1:
                    pl.semaphore_wait(credit_b, 1)
                rdma_b = pltpu.make_async_remote_copy(
                    src_ref=comm_b.at[cur], dst_ref=comm_b.at[nxt],
                    send_sem=send_b.at[cur], recv_sem=recv_b.at[nxt],
                    device_id=(left,), device_id_type=pl.DeviceIdType.MESH,
                )
                rdma_b.start()

            compute(comm_a, cur, lax.rem(my - h + N_DEV, N_DEV))
            if h >= 1:
                compute(comm_b, cur, lax.rem(my + h, N_DEV))

            rdma_a.wait()
            pl.semaphore_signal(
                credit_a, inc=1,
                device_id=(left,), device_id_type=pl.DeviceIdType.MESH,
            )
            if b_active:
                rdma_b.wait()
                pl.semaphore_signal(
                    credit_b, inc=1,
                    device_id=(right,), device_id_type=pl.DeviceIdType.MESH,
                )

        compute(comm_a, A_HOPS % 2, lax.rem(my - A_HOPS + N_DEV, N_DEV))
        pl.semaphore_wait(credit_a, 1)
        pl.semaphore_wait(credit_b, 1)

    return pl.pallas_call(
        body,
        out_shape=jax.ShapeDtypeStruct((T, H), jnp.float32),
        in_specs=[pl.BlockSpec(memory_space=pltpu.VMEM)] * 4,
        out_specs=pl.BlockSpec(memory_space=pltpu.VMEM),
        scratch_shapes=[
            pltpu.VMEM((T, D), jnp.bfloat16),
            pltpu.VMEM((T, E), jnp.float32),
            pltpu.VMEM((2, E_LOC, D, H), jnp.bfloat16),
            pltpu.VMEM((2, E_LOC, D, H), jnp.bfloat16),
            pltpu.SemaphoreType.DMA((2,)),
            pltpu.SemaphoreType.DMA((2,)),
            pltpu.SemaphoreType.DMA((2,)),
            pltpu.SemaphoreType.DMA((2,)),
            pltpu.SemaphoreType.REGULAR,
            pltpu.SemaphoreType.REGULAR,
        ],
        compiler_params=pltpu.CompilerParams(collective_id=0),
    )(x, router_W, route_idx, expert_W)


# device time: 231198 ns/iter; 2.4909x vs baseline; 1.7317x over previous
import jax
import jax.numpy as jnp
import numpy as np
from jax import lax
from jax.experimental import pallas as pl
from jax.experimental.pallas import tpu as pltpu

N_DEV = 32
A_HOPS = N_DEV // 2
B_HOPS = N_DEV - 1 - A_HOPS


def _ring_tables():
    plane = [(0, 0), (1, 0), (1, 1), (0, 1), (0, 2), (1, 2), (1, 3), (0, 3)]
    coords_of = {}
    for k in range(N_DEV):
        px, py = plane[k % 8]
        coords_of[k] = (px, py, k // 8)
    logical_of = {v: k for k, v in coords_of.items()}

    yz = [(0, 0), (1, 0), (2, 0), (3, 0), (3, 1), (2, 1), (1, 1), (0, 1),
          (0, 2), (1, 2), (2, 2), (3, 2), (3, 3), (2, 3), (1, 3), (0, 3)]
    cyc_coords = [(0, y, z) for (y, z) in yz] + [(1, y, z) for (y, z) in reversed(yz)]
    cycle = [logical_of[c] for c in cyc_coords]

    pos = [0] * N_DEV
    nxt = [0] * N_DEV
    prv = [0] * N_DEV
    for j, l in enumerate(cycle):
        pos[l] = j
        nxt[l] = cycle[(j + 1) % N_DEV]
        prv[l] = cycle[(j - 1) % N_DEV]
    to_arr = lambda t: np.asarray(t, np.int32).reshape(1, N_DEV)
    return to_arr(cycle), to_arr(pos), to_arr(nxt), to_arr(prv)


_CYCLE, _POS, _NEXT, _PREV = _ring_tables()


def kernel(x, router_W, route_idx, expert_W):
    T, D = x.shape
    E_LOC, _, H = expert_W.shape
    E = N_DEV * E_LOC

    def body(x_ref, rw_ref, idx_ref, ew_ref,
             cyc_ref, pos_ref, nxt_ref, prv_ref, out_ref,
             xbf_ref, w_ref, comm_a, comm_b,
             send_a, recv_a, send_b, recv_b, credit_a, credit_b):
        my = lax.axis_index("i")

        def lut(idx, table_ref):
            i2 = lax.broadcasted_iota(jnp.int32, (1, N_DEV), 1)
            return jnp.sum(jnp.where(i2 == idx, table_ref[:, :], 0))

        pos_my = lut(my, pos_ref)
        right = lut(my, nxt_ref)
        left = lut(my, prv_ref)

        barrier_sem = pltpu.get_barrier_semaphore()
        for nbr in (left, right):
            pl.semaphore_signal(
                barrier_sem, inc=1,
                device_id=(nbr,), device_id_type=pl.DeviceIdType.MESH,
            )
        pl.semaphore_wait(barrier_sem, 2)

        scores = jnp.dot(x_ref[:, :], rw_ref[:, :],
                         preferred_element_type=jnp.float32)
        e_iota = lax.broadcasted_iota(jnp.int32, (T, E), 1)
        oh0 = (e_iota == idx_ref[:, 0:1]).astype(jnp.float32)
        oh1 = (e_iota == idx_ref[:, 1:2]).astype(jnp.float32)
        s0 = jnp.sum(scores * oh0, axis=1, keepdims=True)
        s1 = jnp.sum(scores * oh1, axis=1, keepdims=True)
        m = jnp.maximum(s0, s1)
        g0 = jnp.exp(s0 - m)
        g1 = jnp.exp(s1 - m)
        den = g0 + g1
        w_ref[:, :] = oh0 * (g0 / den) + oh1 * (g1 / den)

        xbf_ref[:, :] = x_ref[:, :].astype(jnp.bfloat16)
        blk = ew_ref[:, :, :].astype(jnp.bfloat16)
        comm_a[0, :, :, :] = blk
        comm_b[0, :, :, :] = blk
        out_ref[:, :] = jnp.zeros((T, H), jnp.float32)

        def compute(comm_ref, slot, origin):
            r_iota = lax.broadcasted_iota(jnp.int32, (E, E_LOC), 0)
            c_iota = lax.broadcasted_iota(jnp.int32, (E, E_LOC), 1)
            sel = (r_iota == E_LOC * origin + c_iota).astype(jnp.float32)
            wb = jnp.dot(w_ref[:, :], sel,
                         preferred_element_type=jnp.float32)
            for k in range(E_LOC):
                part = jnp.dot(xbf_ref[:, :], comm_ref[slot, k],
                               preferred_element_type=jnp.float32)
                out_ref[:, :] += wb[:, k:k + 1] * part

        for h in range(A_HOPS):
            cur = h % 2
            nxt = (h + 1) % 2
            b_active = h < B_HOPS

            if h >= 1:
                pl.semaphore_wait(credit_a, 1)
            rdma_a = pltpu.make_async_remote_copy(
                src_ref=comm_a.at[cur], dst_ref=comm_a.at[nxt],
                send_sem=send_a.at[cur], recv_sem=recv_a.at[nxt],
                device_id=(right,), device_id_type=pl.DeviceIdType.MESH,
            )
            rdma_a.start()

            if b_active:
                if h >= 1:
                    pl.semaphore_wait(credit_b, 1)
                rdma_b = pltpu.make_async_remote_copy(
                    src_ref=comm_b.at[cur], dst_ref=comm_b.at[nxt],
                    send_sem=send_b.at[cur], recv_sem=recv_b.at[nxt],
                    device_id=(left,), device_id_type=pl.DeviceIdType.MESH,
                )
                rdma_b.start()

            compute(comm_a, cur, lut(lax.rem(pos_my - h + N_DEV, N_DEV), cyc_ref))
            if h >= 1:
                compute(comm_b, cur, lut(lax.rem(pos_my + h, N_DEV), cyc_ref))

            rdma_a.wait()
            pl.semaphore_signal(
                credit_a, inc=1,
                device_id=(left,), device_id_type=pl.DeviceIdType.MESH,
            )
            if b_active:
                rdma_b.wait()
                pl.semaphore_signal(
                    credit_b, inc=1,
                    device_id=(right,), device_id_type=pl.DeviceIdType.MESH,
                )

        compute(comm_a, A_HOPS % 2, lut(lax.rem(pos_my - A_HOPS + N_DEV, N_DEV), cyc_ref))
        pl.semaphore_wait(credit_a, 1)
        pl.semaphore_wait(credit_b, 1)

    return pl.pallas_call(
        body,
        out_shape=jax.ShapeDtypeStruct((T, H), jnp.float32),
        in_specs=[pl.BlockSpec(memory_space=pltpu.VMEM)] * 8,
        out_specs=pl.BlockSpec(memory_space=pltpu.VMEM),
        scratch_shapes=[
            pltpu.VMEM((T, D), jnp.bfloat16),
            pltpu.VMEM((T, E), jnp.float32),
            pltpu.VMEM((2, E_LOC, D, H), jnp.bfloat16),
            pltpu.VMEM((2, E_LOC, D, H), jnp.bfloat16),
            pltpu.SemaphoreType.DMA((2,)),
            pltpu.SemaphoreType.DMA((2,)),
            pltpu.SemaphoreType.DMA((2,)),
            pltpu.SemaphoreType.DMA((2,)),
            pltpu.SemaphoreType.REGULAR,
            pltpu.SemaphoreType.REGULAR,
        ],
        compiler_params=pltpu.CompilerParams(collective_id=0),
    )(x, router_W, route_idx, expert_W,
      jnp.asarray(_CYCLE), jnp.asarray(_POS),
      jnp.asarray(_NEXT), jnp.asarray(_PREV))
